# Initial kernel scaffold; baseline (speedup 1.0000x reference)
import jax
import jax.numpy as jnp
from jax.experimental import pallas as pl


def kernel(x, emb_weight):
    raise NotImplementedError("write your pallas kernel here")


# speedup vs baseline: 1.6957x; 1.6957x over previous
import jax
import jax.numpy as jnp
from jax import lax
from jax.experimental import pallas as pl
from jax.experimental.pallas import tpu as pltpu
from jax.experimental.pallas import tpu_sc as plsc

V_DIM = 1000000
EMB = 32
BATCH = 16384
CTX = 20
NC, NS = 2, 16
NW = NC * NS
S_PER_W = BATCH // NW
CHUNK = 128
N_CHUNKS = S_PER_W // CHUNK
ROWS = CHUNK * CTX
INV_CTX = float(1.0 / CTX)


def _sc_body(x_hbm, tab_hbm, out_hbm, idx_v, rows_v, out_v, sem):
    wid = lax.axis_index("s") * NC + lax.axis_index("c")
    base_out = wid * S_PER_W

    def chunk_body(ci, carry):
        off_out = base_out + ci * CHUNK
        off_idx = off_out * CTX
        pltpu.sync_copy(x_hbm.at[pl.ds(off_idx, ROWS)], idx_v)
        pltpu.async_copy(tab_hbm.at[idx_v], rows_v, sem).wait()

        def out_body(o, c2):
            base = o * CTX
            for h in range(EMB // 16):
                sl = pl.ds(h * 16, 16)
                vals = [rows_v[base + c, sl] for c in range(CTX)]
                while len(vals) > 1:
                    vals = [a + b for a, b in zip(vals[::2], vals[1::2])] + (
                        [vals[-1]] if len(vals) % 2 else [])
                out_v[o, sl] = vals[0] * INV_CTX
            return c2

        lax.fori_loop(0, CHUNK, out_body, 0)
        pltpu.sync_copy(out_v, out_hbm.at[pl.ds(off_out, CHUNK)])
        return carry

    lax.fori_loop(0, N_CHUNKS, chunk_body, 0)


@jax.jit
def _cbow(x_flat, tab):
    mesh = plsc.VectorSubcoreMesh(core_axis_name="c", subcore_axis_name="s")
    f = pl.kernel(
        _sc_body,
        out_type=jax.ShapeDtypeStruct((BATCH, EMB), jnp.float32),
        mesh=mesh,
        scratch_types=[
            pltpu.VMEM((ROWS,), jnp.int32),
            pltpu.VMEM((ROWS, EMB), jnp.float32),
            pltpu.VMEM((CHUNK, EMB), jnp.float32),
            pltpu.SemaphoreType.DMA,
        ],
        compiler_params=pltpu.CompilerParams(use_tc_tiling_on_sc=False),
    )
    return f(x_flat, tab)


def kernel(x, emb_weight):
    return _cbow(x.reshape(-1), emb_weight)
